# speedup vs baseline: 1.4805x; 1.4805x over previous
import functools

import jax
import jax.numpy as jnp
from jax import lax
from jax.experimental import pallas as pl
from jax.experimental.pallas import tpu as pltpu
from jax.experimental.pallas import tpu_sc as plsc

N = 10000
E = 160000
EPALL = 40000
ROWB = 2000


def _mm_body(x_ref, w_ref, o_ref):
    o_ref[...] = jnp.dot(x_ref[...], w_ref[...],
                         preferred_element_type=jnp.float32)


def _matmul(x, w):
    n, k = x.shape
    m = w.shape[1]
    return pl.pallas_call(
        _mm_body,
        grid=(n // ROWB,),
        in_specs=[pl.BlockSpec((ROWB, k), lambda i: (i, 0)),
                  pl.BlockSpec((k, m), lambda i: (0, 0))],
        out_specs=pl.BlockSpec((ROWB, m), lambda i: (i, 0)),
        out_shape=jax.ShapeDtypeStruct((n, m), jnp.float32),
    )(x, w)


def _combine_mm_body(s_ref, a0_ref, a1_ref, deg_ref, b_ref, w_ref, o_ref, *,
                     relu):
    agg = jnp.concatenate([a0_ref[...], a1_ref[...]], axis=1)
    deg = jnp.maximum(deg_ref[...][:, 0:1], 1.0)
    h = s_ref[...] + agg / deg + b_ref[...]
    if relu:
        h = jnp.maximum(h, 0.0)
    o_ref[...] = jnp.dot(h, w_ref[...], preferred_element_type=jnp.float32)


def _combine_mm(s, a0, a1, deg16, b, w, relu):
    n, dh = s.shape
    dc = a0.shape[1]
    m = w.shape[1]
    return pl.pallas_call(
        functools.partial(_combine_mm_body, relu=relu),
        grid=(n // ROWB,),
        in_specs=[pl.BlockSpec((ROWB, dh), lambda i: (i, 0)),
                  pl.BlockSpec((ROWB, dc), lambda i: (i, 0)),
                  pl.BlockSpec((ROWB, dc), lambda i: (i, 0)),
                  pl.BlockSpec((ROWB, 16), lambda i: (i, 0)),
                  pl.BlockSpec((1, dh), lambda i: (0, 0)),
                  pl.BlockSpec((dh, m), lambda i: (0, 0))],
        out_specs=pl.BlockSpec((ROWB, m), lambda i: (i, 0)),
        out_shape=jax.ShapeDtypeStruct((n, m), jnp.float32),
    )(s, a0, a1, deg16, b.reshape(1, dh), w)


def _score_body(z_ref, bp1_ref, wp2_ref, bp2_ref, o_ref):
    z = jnp.maximum(z_ref[...] + bp1_ref[...], 0.0)
    o_ref[...] = jnp.sum(z * wp2_ref[...], axis=1, keepdims=True) + bp2_ref[0, 0]


def _score(z, bp1, wp2, bp2):
    n, d = z.shape
    return pl.pallas_call(
        _score_body,
        grid=(n // ROWB,),
        in_specs=[pl.BlockSpec((ROWB, d), lambda i: (i, 0)),
                  pl.BlockSpec((1, d), lambda i: (0, 0)),
                  pl.BlockSpec((1, d), lambda i: (0, 0)),
                  pl.BlockSpec((1, 1), lambda i: (0, 0))],
        out_specs=pl.BlockSpec((ROWB, 1), lambda i: (i, 0)),
        out_shape=jax.ShapeDtypeStruct((n, 1), jnp.float32),
    )(z, bp1.reshape(1, d), wp2.reshape(1, d), bp2.reshape(1, 1))


def _seg_mean_parts(p, src, dst):
    d = p.shape[1]
    s = jax.ops.segment_sum(jnp.take(p, src, axis=0), dst, num_segments=N)
    deg = jax.ops.segment_sum(jnp.ones((src.shape[0],), jnp.float32), dst,
                              num_segments=N)
    deg16 = jnp.broadcast_to(deg[:, None], (N, 16))
    return s[:, :d // 2], s[:, d // 2:], deg16


def _gather_add(a, b, e0, e1):
    return jnp.take(a, e0, axis=0) + jnp.take(b, e1, axis=0)


def kernel(x, edge_index, pos_edge_index, neg_edge_index, W_self1, W_neigh1,
           b1, W_self2, W_neigh2, b2, Wp1, bp1, Wp2, bp2):
    src = edge_index[0]
    dst = edge_index[1]

    cat1 = _matmul(x, jnp.concatenate([W_self1, W_neigh1], axis=1))
    s1 = cat1[:, :256]
    p1 = cat1[:, 256:]

    a10, a11, deg16 = _seg_mean_parts(p1, src, dst)

    cat2 = _combine_mm(s1, a10, a11, deg16, b1,
                       jnp.concatenate([W_self2, W_neigh2], axis=1), True)
    s2 = cat2[:, :64]
    p2 = cat2[:, 64:]

    a20, a21, _ = _seg_mean_parts(p2, src, dst)

    ab = _combine_mm(s2, a20, a21, deg16, b2,
                     jnp.concatenate([Wp1[:64], Wp1[64:]], axis=1), False)
    asrc = ab[:, :64]
    bdst = ab[:, 64:]

    eall = jnp.concatenate([pos_edge_index, neg_edge_index], axis=1)
    z = _gather_add(asrc, bdst, eall[0], eall[1])

    scores = _score(z, bp1, Wp2, bp2)[:, 0]
    return scores[:20000], scores[20000:]


# speedup vs baseline: 3.3520x; 2.2641x over previous
import functools

import jax
import jax.numpy as jnp
from jax import lax
from jax.experimental import pallas as pl
from jax.experimental.pallas import tpu as pltpu
from jax.experimental.pallas import tpu_sc as plsc

N = 10000
E = 160000
EPALL = 40000
ROWB = 2000


def _mm_body(x_ref, w_ref, o_ref):
    o_ref[...] = jnp.dot(x_ref[...], w_ref[...],
                         preferred_element_type=jnp.float32)


def _matmul(x, w):
    n, k = x.shape
    m = w.shape[1]
    return pl.pallas_call(
        _mm_body,
        grid=(n // ROWB,),
        in_specs=[pl.BlockSpec((ROWB, k), lambda i: (i, 0)),
                  pl.BlockSpec((k, m), lambda i: (0, 0))],
        out_specs=pl.BlockSpec((ROWB, m), lambda i: (i, 0)),
        out_shape=jax.ShapeDtypeStruct((n, m), jnp.float32),
    )(x, w)


def _combine_mm_body(*refs, relu, naggs):
    s_ref = refs[0]
    agg_refs = refs[1:1 + naggs]
    deg_ref, b_ref, w_ref, o_ref = refs[1 + naggs:]
    agg = jnp.concatenate([a[...] for a in agg_refs], axis=1)
    deg = jnp.maximum(deg_ref[...][:, 0:1], 1.0)
    h = s_ref[...] + agg / deg + b_ref[...]
    if relu:
        h = jnp.maximum(h, 0.0)
    o_ref[...] = jnp.dot(h, w_ref[...], preferred_element_type=jnp.float32)


def _combine_mm(s, aggs, deg16, b, w, relu):
    n, dh = s.shape
    m = w.shape[1]
    agg_specs = [pl.BlockSpec((ROWB, a.shape[1]), lambda i: (i, 0))
                 for a in aggs]
    return pl.pallas_call(
        functools.partial(_combine_mm_body, relu=relu, naggs=len(aggs)),
        grid=(n // ROWB,),
        in_specs=[pl.BlockSpec((ROWB, dh), lambda i: (i, 0))] + agg_specs +
                 [pl.BlockSpec((ROWB, 16), lambda i: (i, 0)),
                  pl.BlockSpec((1, dh), lambda i: (0, 0)),
                  pl.BlockSpec((dh, m), lambda i: (0, 0))],
        out_specs=pl.BlockSpec((ROWB, m), lambda i: (i, 0)),
        out_shape=jax.ShapeDtypeStruct((n, m), jnp.float32),
    )(s, *aggs, deg16, b.reshape(1, dh), w)


def _score_body(za_ref, zb_ref, bp1_ref, wp2_ref, bp2_ref, o_ref):
    z = jnp.maximum(za_ref[...] + zb_ref[...] + bp1_ref[...], 0.0)
    o_ref[...] = jnp.sum(z * wp2_ref[...], axis=1, keepdims=True) + bp2_ref[0, 0]


def _score(za, zb, bp1, wp2, bp2, rowb):
    n, d = za.shape
    return pl.pallas_call(
        _score_body,
        grid=(n // rowb,),
        in_specs=[pl.BlockSpec((rowb, d), lambda i: (i, 0)),
                  pl.BlockSpec((rowb, d), lambda i: (i, 0)),
                  pl.BlockSpec((1, d), lambda i: (0, 0)),
                  pl.BlockSpec((1, d), lambda i: (0, 0)),
                  pl.BlockSpec((1, 1), lambda i: (0, 0))],
        out_specs=pl.BlockSpec((rowb, 1), lambda i: (i, 0)),
        out_shape=jax.ShapeDtypeStruct((n, 1), jnp.float32),
    )(za, zb, bp1.reshape(1, d), wp2.reshape(1, d), bp2.reshape(1, 1))


NC = 2
NS = 16
CH = 80
EPT = E // NS
NCH = EPT // CH
NPAD = 10112
RPT = NPAD // NS
EPAD = 40960
EPW = EPAD // (NC * NS)


def _sc_mesh():
    return plsc.VectorSubcoreMesh(core_axis_name="c", subcore_axis_name="s",
                                  num_cores=NC, num_subcores=NS)


def _make_seg_kernel(dc, with_deg):
    out_type = [jax.ShapeDtypeStruct((NPAD, dc), jnp.float32),
                jax.ShapeDtypeStruct((NPAD, dc), jnp.float32)]
    scratch = [pltpu.VMEM_SHARED((NPAD, dc), jnp.float32),
               pltpu.VMEM((RPT, dc), jnp.float32),
               pltpu.VMEM((CH,), jnp.int32),
               pltpu.VMEM((CH,), jnp.int32),
               pltpu.VMEM((CH, dc), jnp.float32),
               pltpu.SemaphoreType.DMA]
    if with_deg:
        out_type.append(jax.ShapeDtypeStruct((NPAD, 16), jnp.float32))
        scratch += [pltpu.VMEM_SHARED((NPAD, 16), jnp.float32),
                    pltpu.VMEM((RPT, 16), jnp.float32),
                    pltpu.VMEM((CH, 16), jnp.float32)]

    def body(pa, pb, srci, dsti, zrows, *rest):
        if with_deg:
            (zdeg, ones_in, outa, outb, deg_out,
             accum, zbuf, srcv, dstv, rows, sem, degacc, dbuf, onesv) = rest
        else:
            outa, outb, accum, zbuf, srcv, dstv, rows, sem = rest
        s = lax.axis_index("s")
        c = lax.axis_index("c")
        r0 = pl.multiple_of(s * RPT, RPT)

        pltpu.sync_copy(zrows, zbuf)
        pltpu.sync_copy(zbuf, accum.at[pl.ds(r0, RPT)])
        if with_deg:
            @pl.when(c == 0)
            def _():
                pltpu.sync_copy(ones_in, onesv)
                pltpu.sync_copy(zdeg, dbuf)
                pltpu.sync_copy(dbuf, degacc.at[pl.ds(r0, RPT)])
        plsc.subcore_barrier()

        def run(p_hbm, do_deg):
            def chunk(i, carry):
                base = pl.multiple_of(s * EPT + i * CH, CH)
                pltpu.sync_copy(srci.at[pl.ds(base, CH)], srcv)
                pltpu.sync_copy(dsti.at[pl.ds(base, CH)], dstv)
                pltpu.async_copy(p_hbm.at[srcv], rows, sem).wait()
                pltpu.sync_copy(rows, accum.at[dstv], add=True)
                if do_deg:
                    pltpu.sync_copy(onesv, degacc.at[dstv], add=True)
                return carry
            lax.fori_loop(0, NCH, chunk, 0)

        @pl.when(c == 0)
        def _():
            run(pa, with_deg)

        @pl.when(c == 1)
        def _():
            run(pb, False)

        plsc.subcore_barrier()

        pltpu.sync_copy(accum.at[pl.ds(r0, RPT)], zbuf)

        @pl.when(c == 0)
        def _():
            pltpu.sync_copy(zbuf, outa.at[pl.ds(r0, RPT)])
            if with_deg:
                pltpu.sync_copy(degacc.at[pl.ds(r0, RPT)], dbuf)
                pltpu.sync_copy(dbuf, deg_out.at[pl.ds(r0, RPT)])

        @pl.when(c == 1)
        def _():
            pltpu.sync_copy(zbuf, outb.at[pl.ds(r0, RPT)])

    return pl.kernel(body, out_type=tuple(out_type), mesh=_sc_mesh(),
                     scratch_types=tuple(scratch),
                     compiler_params=pltpu.CompilerParams(
                         use_tc_tiling_on_sc=False))


def _seg_mean_parts(pa, pb, src, dst, with_deg, deg_unused=None):
    dc = pa.shape[1]
    zrows = jnp.zeros((RPT, dc), jnp.float32)
    if with_deg:
        zdeg = jnp.zeros((RPT, 16), jnp.float32)
        ones = jnp.ones((CH, 16), jnp.float32)
        return _make_seg_kernel(dc, True)(pa, pb, src, dst, zrows, zdeg, ones)
    return _make_seg_kernel(dc, False)(pa, pb, src, dst, zrows)


def _make_pair_gather():
    out_type = (jax.ShapeDtypeStruct((EPAD, 64), jnp.float32),
                jax.ShapeDtypeStruct((EPAD, 64), jnp.float32))
    scratch = (pltpu.VMEM((CH,), jnp.int32),
               pltpu.VMEM((CH, 64), jnp.float32),
               pltpu.SemaphoreType.DMA)

    def body(atab, btab, e0, e1, za, zb, idxv, rows, sem):
        s = lax.axis_index("s")
        c = lax.axis_index("c")
        w = c * NS + s

        def chunk(i, carry):
            base = pl.multiple_of(w * EPW + i * CH, CH)
            pltpu.sync_copy(e0.at[pl.ds(base, CH)], idxv)
            pltpu.async_copy(atab.at[idxv], rows, sem).wait()
            pltpu.sync_copy(rows, za.at[pl.ds(base, CH)])
            pltpu.sync_copy(e1.at[pl.ds(base, CH)], idxv)
            pltpu.async_copy(btab.at[idxv], rows, sem).wait()
            pltpu.sync_copy(rows, zb.at[pl.ds(base, CH)])
            return carry
        lax.fori_loop(0, EPW // CH, chunk, 0)

    return pl.kernel(body, out_type=out_type, mesh=_sc_mesh(),
                     scratch_types=scratch,
                     compiler_params=pltpu.CompilerParams(
                         use_tc_tiling_on_sc=False))


def kernel(x, edge_index, pos_edge_index, neg_edge_index, W_self1, W_neigh1,
           b1, W_self2, W_neigh2, b2, Wp1, bp1, Wp2, bp2):
    src = edge_index[0]
    dst = edge_index[1]

    cat1 = _matmul(x, jnp.concatenate([W_self1, W_neigh1], axis=1))
    s1 = cat1[:, :256]

    a10, a11 = _seg_mean_parts(cat1[:, 256:320], cat1[:, 320:384],
                               src, dst, False)
    a12, a13, deg16 = _seg_mean_parts(cat1[:, 384:448], cat1[:, 448:512],
                                      src, dst, True)

    cat2 = _combine_mm(s1, [a10, a11, a12, a13], deg16, b1,
                       jnp.concatenate([W_self2, W_neigh2], axis=1), True)
    s2 = cat2[:, :64]
    p2a = cat2[:, 64:96]
    p2b = cat2[:, 96:]

    a20, a21 = _seg_mean_parts(p2a, p2b, src, dst, False)

    ab = _combine_mm(s2, [a20, a21], deg16, b2,
                     jnp.concatenate([Wp1[:64], Wp1[64:]], axis=1), False)
    asrc = ab[:, :64]
    bdst = ab[:, 64:]

    eall = jnp.pad(
        jnp.concatenate([pos_edge_index, neg_edge_index], axis=1),
        ((0, 0), (0, EPAD - EPALL)))
    za, zb = _make_pair_gather()(asrc, bdst, eall[0], eall[1])

    scores = _score(za, zb, bp1, Wp2, bp2, 2048)[:, 0]
    return scores[:20000], scores[20000:EPALL]


# speedup vs baseline: 6.9124x; 2.0622x over previous
import functools

import jax
import jax.numpy as jnp
from jax import lax
from jax.experimental import pallas as pl
from jax.experimental.pallas import tpu as pltpu
from jax.experimental.pallas import tpu_sc as plsc

N = 10000
E = 160000
EPALL = 40000
ROWB = 2000


def _mm_body(x_ref, w_ref, o_ref):
    o_ref[...] = jnp.dot(x_ref[...], w_ref[...],
                         preferred_element_type=jnp.float32)


def _matmul(x, w):
    n, k = x.shape
    m = w.shape[1]
    return pl.pallas_call(
        _mm_body,
        grid=(n // ROWB,),
        in_specs=[pl.BlockSpec((ROWB, k), lambda i: (i, 0)),
                  pl.BlockSpec((k, m), lambda i: (0, 0))],
        out_specs=pl.BlockSpec((ROWB, m), lambda i: (i, 0)),
        out_shape=jax.ShapeDtypeStruct((n, m), jnp.float32),
    )(x, w)


def _combine_mm_body(*refs, relu, naggs):
    s_ref = refs[0]
    agg_refs = refs[1:1 + naggs]
    deg_ref, b_ref, w_ref, o_ref = refs[1 + naggs:]
    agg = jnp.concatenate([a[...] for a in agg_refs], axis=1)
    deg = jnp.maximum(deg_ref[...][:, 0:1], 1.0)
    h = s_ref[...] + agg / deg + b_ref[...]
    if relu:
        h = jnp.maximum(h, 0.0)
    o_ref[...] = jnp.dot(h, w_ref[...], preferred_element_type=jnp.float32)


def _combine_mm(s, aggs, deg16, b, w, relu):
    n, dh = s.shape
    m = w.shape[1]
    agg_specs = [pl.BlockSpec((ROWB, a.shape[1]), lambda i: (i, 0))
                 for a in aggs]
    return pl.pallas_call(
        functools.partial(_combine_mm_body, relu=relu, naggs=len(aggs)),
        grid=(n // ROWB,),
        in_specs=[pl.BlockSpec((ROWB, dh), lambda i: (i, 0))] + agg_specs +
                 [pl.BlockSpec((ROWB, 16), lambda i: (i, 0)),
                  pl.BlockSpec((1, dh), lambda i: (0, 0)),
                  pl.BlockSpec((dh, m), lambda i: (0, 0))],
        out_specs=pl.BlockSpec((ROWB, m), lambda i: (i, 0)),
        out_shape=jax.ShapeDtypeStruct((n, m), jnp.float32),
    )(s, *aggs, deg16, b.reshape(1, dh), w)


def _score_body(za_ref, zb_ref, bp1_ref, wp2_ref, bp2_ref, o_ref):
    z = jnp.maximum(za_ref[...] + zb_ref[...] + bp1_ref[...], 0.0)
    o_ref[...] = jnp.sum(z * wp2_ref[...], axis=1, keepdims=True) + bp2_ref[0, 0]


def _score(za, zb, bp1, wp2, bp2, rowb):
    n, d = za.shape
    return pl.pallas_call(
        _score_body,
        grid=(n // rowb,),
        in_specs=[pl.BlockSpec((rowb, d), lambda i: (i, 0)),
                  pl.BlockSpec((rowb, d), lambda i: (i, 0)),
                  pl.BlockSpec((1, d), lambda i: (0, 0)),
                  pl.BlockSpec((1, d), lambda i: (0, 0)),
                  pl.BlockSpec((1, 1), lambda i: (0, 0))],
        out_specs=pl.BlockSpec((rowb, 1), lambda i: (i, 0)),
        out_shape=jax.ShapeDtypeStruct((n, 1), jnp.float32),
    )(za, zb, bp1.reshape(1, d), wp2.reshape(1, d), bp2.reshape(1, 1))


NC = 2
NS = 16
CH = 80
EPT = E // NS
NCH = EPT // CH
NB = 5
Q = 4
NPAD = 10112
RPT = NPAD // NS
EPAD = 40960
EPW = EPAD // (NC * NS)


def _sc_mesh():
    return plsc.VectorSubcoreMesh(core_axis_name="c", subcore_axis_name="s",
                                  num_cores=NC, num_subcores=NS)


def _make_seg_kernel(dc, with_deg):
    out_type = [jax.ShapeDtypeStruct((NPAD, dc), jnp.float32),
                jax.ShapeDtypeStruct((NPAD, dc), jnp.float32)]
    scratch = [pltpu.VMEM_SHARED((NPAD, dc), jnp.float32),
               pltpu.VMEM((RPT // Q, dc), jnp.float32),
               pltpu.VMEM((NCH, CH), jnp.int32),
               pltpu.VMEM((NB, CH, dc), jnp.float32),
               pltpu.SemaphoreType.DMA,
               pltpu.SemaphoreType.DMA,
               pltpu.SemaphoreType.DMA]
    scratch += [pltpu.VMEM((CH,), jnp.int32) for _ in range(NB)]
    if with_deg:
        out_type.append(jax.ShapeDtypeStruct((NPAD, 16), jnp.float32))
        scratch += [pltpu.VMEM_SHARED((NPAD, 16), jnp.float32),
                    pltpu.VMEM((RPT // Q, 16), jnp.float32),
                    pltpu.VMEM((CH, 16), jnp.float32),
                    pltpu.SemaphoreType.DMA]

    def body(pa, pb, src2d, dst1d, zrows, *rest):
        if with_deg:
            zdeg, ones_in, outa, outb, deg_out = rest[:5]
            rest = rest[5:]
        else:
            outa, outb = rest[:2]
            rest = rest[2:]
        accum, zbuf, sidx, rows, gsem, ssem, isem = rest[:7]
        didxb = list(rest[7:7 + NB])
        if with_deg:
            degacc, dbuf, onesv, dsem = rest[7 + NB:]
        s = lax.axis_index("s")
        c = lax.axis_index("c")
        r0 = pl.multiple_of(s * RPT, RPT)

        pltpu.sync_copy(src2d.at[pl.ds(s * NCH, NCH)], sidx)
        pltpu.sync_copy(zrows, zbuf)
        for q in range(Q):
            pltpu.sync_copy(zbuf, accum.at[pl.ds(r0 + q * (RPT // Q),
                                                 RPT // Q)])
        if with_deg:
            @pl.when(c == 0)
            def _():
                pltpu.sync_copy(ones_in, onesv)
                pltpu.sync_copy(zdeg, dbuf)
                for q in range(Q):
                    pltpu.sync_copy(dbuf, degacc.at[pl.ds(r0 + q * (RPT // Q),
                                                          RPT // Q)])
        plsc.subcore_barrier()

        def run(p_hbm, do_deg):
            def sup(g, carry):
                c0 = g * NB
                idxd = []
                for b in range(NB):
                    base = pl.multiple_of(s * EPT + (c0 + b) * CH, CH)
                    idxd.append(pltpu.async_copy(
                        dst1d.at[pl.ds(base, CH)], didxb[b], isem))
                gd = [pltpu.async_copy(p_hbm.at[sidx.at[c0 + b]],
                                       rows.at[b], gsem)
                      for b in range(NB)]
                for d in idxd + gd:
                    d.wait()
                for b in range(NB):
                    sd = pltpu.async_copy(rows.at[b], accum.at[didxb[b]],
                                          ssem, add=True)
                    if do_deg:
                        pltpu.async_copy(onesv, degacc.at[didxb[b]],
                                         dsem, add=True).wait()
                    sd.wait()
                return carry
            lax.fori_loop(0, NCH // NB, sup, 0)

        @pl.when(c == 0)
        def _():
            run(pa, with_deg)

        @pl.when(c == 1)
        def _():
            run(pb, False)

        plsc.subcore_barrier()

        for q in range(Q):
            qs = pl.ds(r0 + q * (RPT // Q), RPT // Q)
            pltpu.sync_copy(accum.at[qs], zbuf)

            @pl.when(c == 0)
            def _():
                pltpu.sync_copy(zbuf, outa.at[qs])

            @pl.when(c == 1)
            def _():
                pltpu.sync_copy(zbuf, outb.at[qs])
        if with_deg:
            @pl.when(c == 0)
            def _():
                for q in range(Q):
                    qs = pl.ds(r0 + q * (RPT // Q), RPT // Q)
                    pltpu.sync_copy(degacc.at[qs], dbuf)
                    pltpu.sync_copy(dbuf, deg_out.at[qs])

    return pl.kernel(body, out_type=tuple(out_type), mesh=_sc_mesh(),
                     scratch_types=tuple(scratch),
                     compiler_params=pltpu.CompilerParams(
                         use_tc_tiling_on_sc=False))


def _seg_mean_parts(pa, pb, src, dst, with_deg, deg_unused=None):
    dc = pa.shape[1]
    zrows = jnp.zeros((RPT // Q, dc), jnp.float32)
    if with_deg:
        zdeg = jnp.zeros((RPT // Q, 16), jnp.float32)
        ones = jnp.ones((CH, 16), jnp.float32)
        return _make_seg_kernel(dc, True)(pa, pb, src, dst, zrows, zdeg, ones)
    return _make_seg_kernel(dc, False)(pa, pb, src, dst, zrows)


CPW = EPW // CH
NBP = 4


def _make_pair_gather():
    out_type = (jax.ShapeDtypeStruct((EPAD, 64), jnp.float32),
                jax.ShapeDtypeStruct((EPAD, 64), jnp.float32))
    scratch = (pltpu.VMEM((CPW, CH), jnp.int32),
               pltpu.VMEM((CPW, CH), jnp.int32),
               pltpu.VMEM((NBP, CH, 64), jnp.float32),
               pltpu.VMEM((NBP, CH, 64), jnp.float32),
               pltpu.SemaphoreType.DMA,
               pltpu.SemaphoreType.DMA)

    def body(atab, btab, e0, e1, za, zb, i0, i1, ra, rb, gsem, wsem):
        s = lax.axis_index("s")
        c = lax.axis_index("c")
        w = c * NS + s
        pltpu.sync_copy(e0.at[pl.ds(w * CPW, CPW)], i0)
        pltpu.sync_copy(e1.at[pl.ds(w * CPW, CPW)], i1)

        def sup(g, carry):
            c0 = g * NBP
            gd = ([pltpu.async_copy(atab.at[i0.at[c0 + b]], ra.at[b], gsem)
                   for b in range(NBP)] +
                  [pltpu.async_copy(btab.at[i1.at[c0 + b]], rb.at[b], gsem)
                   for b in range(NBP)])
            for d in gd:
                d.wait()
            wd = []
            for b in range(NBP):
                base = pl.multiple_of(w * EPW + (c0 + b) * CH, CH)
                wd.append(pltpu.async_copy(ra.at[b], za.at[pl.ds(base, CH)],
                                           wsem))
                wd.append(pltpu.async_copy(rb.at[b], zb.at[pl.ds(base, CH)],
                                           wsem))
            for d in wd:
                d.wait()
            return carry
        lax.fori_loop(0, CPW // NBP, sup, 0)

    return pl.kernel(body, out_type=out_type, mesh=_sc_mesh(),
                     scratch_types=scratch,
                     compiler_params=pltpu.CompilerParams(
                         use_tc_tiling_on_sc=False))


def kernel(x, edge_index, pos_edge_index, neg_edge_index, W_self1, W_neigh1,
           b1, W_self2, W_neigh2, b2, Wp1, bp1, Wp2, bp2):
    src = edge_index[0].reshape(E // CH, CH)
    dst = edge_index[1]

    cat1 = _matmul(x, jnp.concatenate([W_self1, W_neigh1], axis=1))
    s1 = cat1[:, :256]

    a10, a11 = _seg_mean_parts(cat1[:, 256:320], cat1[:, 320:384],
                               src, dst, False)
    a12, a13, deg16 = _seg_mean_parts(cat1[:, 384:448], cat1[:, 448:512],
                                      src, dst, True)

    cat2 = _combine_mm(s1, [a10, a11, a12, a13], deg16, b1,
                       jnp.concatenate([W_self2, W_neigh2], axis=1), True)
    s2 = cat2[:, :64]
    p2a = cat2[:, 64:96]
    p2b = cat2[:, 96:]

    a20, a21 = _seg_mean_parts(p2a, p2b, src, dst, False)

    ab = _combine_mm(s2, [a20, a21], deg16, b2,
                     jnp.concatenate([Wp1[:64], Wp1[64:]], axis=1), False)
    asrc = ab[:, :64]
    bdst = ab[:, 64:]

    eall = jnp.pad(
        jnp.concatenate([pos_edge_index, neg_edge_index], axis=1),
        ((0, 0), (0, EPAD - EPALL)))
    za, zb = _make_pair_gather()(asrc, bdst,
                                 eall[0].reshape(EPAD // CH, CH),
                                 eall[1].reshape(EPAD // CH, CH))

    scores = _score(za, zb, bp1, Wp2, bp2, 2048)[:, 0]
    return scores[:20000], scores[20000:EPALL]
